# baseline (speedup 1.0000x reference)
import functools

import jax
import jax.numpy as jnp
from jax import lax
from jax.experimental import pallas as pl
from jax.experimental.pallas import tpu as pltpu
from jax.experimental.pallas import tpu_sc as plsc

BATCH = 16384
EMB = 32
_NC = 2
_NS = 16
_NW = _NC * _NS
_BPW = BATCH // _NW
_NG = _BPW // 16
_PIECE = 512
_TCOLS = 1000001
_TPAD = 1000064


def _stream_table(tab, ids_v, obuf, win0, win1, sem0, sem1):
  gmin = []
  gmax = []
  for g in range(_NG):
    idv = ids_v[pl.ds(g * 16, 16)]
    gmin.append(jnp.min(idv))
    gmax.append(jnp.max(idv))
  lo_all = jnp.minimum(functools.reduce(jnp.minimum, gmin), _TPAD - _PIECE)
  base = (lo_all // 128) * 128
  hi_all = functools.reduce(jnp.maximum, gmax)
  npieces = (hi_all - base) // _PIECE + 1

  def piece_start(p):
    return pl.multiple_of(
        jnp.minimum(base + p * _PIECE, _TPAD - _PIECE), 128)

  def fire(p, win, sem):
    pltpu.async_copy(tab.at[:, pl.ds(piece_start(p), _PIECE)], win, sem)

  def wait(p, win, sem):
    pltpu.make_async_copy(
        tab.at[:, pl.ds(piece_start(p), _PIECE)], win, sem).wait()

  def process(p, win):
    lo = piece_start(p)
    for g in range(_NG):
      @pl.when((gmin[g] < lo + _PIECE) & (gmax[g] >= lo))
      def _(g=g, win=win, lo=lo):
        idv = ids_v[pl.ds(g * 16, 16)]
        col = idv - lo
        msk = (col >= 0) & (col < _PIECE)
        cols = jnp.where(msk, col, 0)
        rows = lax.iota(jnp.int32, 16) + g * 16

        def kbody(k, kv):
          for _ in range(2):
            v = plsc.load_gather(win, [kv, cols], mask=msk)
            plsc.store_scatter(obuf, [rows, kv], v, mask=msk)
            kv = kv + 1
          return kv
        lax.fori_loop(0, EMB // 2, kbody, jnp.zeros((16,), jnp.int32))

  fire(0, win0, sem0)

  def body2(q, carry):
    del carry
    p0 = 2 * q
    p1 = p0 + 1

    @pl.when(p1 < npieces)
    def _():
      fire(p1, win1, sem1)

    wait(p0, win0, sem0)
    process(p0, win0)

    @pl.when(p1 < npieces)
    def _():
      @pl.when(p1 + 1 < npieces)
      def _():
        fire(p1 + 1, win0, sem0)
      wait(p1, win1, sem1)
      process(p1, win1)
    return 0

  lax.fori_loop(0, (npieces + 1) // 2, body2, 0)


def _sc_body(item_t, user_t, dur_t, sids_i, sids_u, pos_i, pos_u, dur_id,
             item_out, user_out, dur_out,
             ids_v, obuf, win0, win1, pos_v, dtab,
             sem0, sem1, ssem):
  wid = lax.axis_index("s") * _NC + lax.axis_index("c")
  sl = pl.ds(wid * _BPW, _BPW)

  def scatter_rows(out):
    copies = [
        pltpu.async_copy(obuf.at[pl.ds(c * 128, 128)],
                         out.at[pos_v.at[c]], ssem)
        for c in range(_BPW // 128)
    ]
    for cp in copies:
      cp.wait()

  pltpu.sync_copy(dur_t, dtab)
  pltpu.sync_copy(dur_id.at[sl], ids_v)

  def dgroup(g, c0):
    idv = ids_v[pl.ds(g * 16, 16)]
    rows = lax.iota(jnp.int32, 16) + g * 16

    def kbody(k, kv):
      for _ in range(4):
        v = plsc.load_gather(dtab, [kv, idv])
        plsc.store_scatter(obuf, [rows, kv], v)
        kv = kv + 1
      return kv
    lax.fori_loop(0, EMB // 4, kbody, jnp.zeros((16,), jnp.int32))
    return c0
  lax.fori_loop(0, _NG, dgroup, 0)
  pltpu.sync_copy(obuf, dur_out.at[sl])

  for tab, sids, pos3, out in ((item_t, sids_i, pos_i, item_out),
                               (user_t, sids_u, pos_u, user_out)):
    pltpu.sync_copy(sids.at[sl], ids_v)
    pltpu.sync_copy(pos3.at[wid], pos_v)
    _stream_table(tab, ids_v, obuf, win0, win1, sem0, sem1)
    scatter_rows(out)


def _mlp_body(item_ref, user_ref, dur_ref, w1_ref, b1_ref, w2_ref, b2_ref,
              w3_ref, b3_ref, wo_ref, bo_ref, out_ref):
  f32 = jnp.float32
  h = jnp.dot(item_ref[:, 0:EMB], w1_ref[0:EMB, :], preferred_element_type=f32)
  h += jnp.dot(user_ref[:, 0:EMB], w1_ref[EMB:2 * EMB, :], preferred_element_type=f32)
  h += jnp.dot(dur_ref[:, 0:EMB], w1_ref[2 * EMB:3 * EMB, :], preferred_element_type=f32)
  h = jnp.maximum(h + b1_ref[...], 0.0)
  h = jnp.maximum(jnp.dot(h, w2_ref[...], preferred_element_type=f32) + b2_ref[...], 0.0)
  h = jnp.maximum(jnp.dot(h, w3_ref[...], preferred_element_type=f32) + b3_ref[...], 0.0)
  z = jnp.dot(h, wo_ref[...], preferred_element_type=f32) + bo_ref[...]
  out_ref[...] = 1.0 / (1.0 + jnp.exp(-z))


def kernel(user_id, item_id, duration, is_training, item_table, user_table,
           dur_table, W1, b1, W2, b2, W3, b3, Wo, bo):
  del is_training

  item_id = item_id.astype(jnp.int32)
  user_id = user_id.astype(jnp.int32)
  duration = duration.astype(jnp.int32)

  item_t = item_table.T
  user_t = user_table.T
  dur_t = jnp.pad(dur_table.T, ((0, 0), (0, 56)))

  iota = lax.iota(jnp.int32, BATCH)

  def prep(ids):
    sids, pos = lax.sort([ids, iota], num_keys=1)
    return sids, pos.reshape(_NW, _BPW // 128, 128)

  sids_i, pos_i = prep(item_id)
  sids_u, pos_u = prep(user_id)

  mesh = plsc.VectorSubcoreMesh(core_axis_name="c", subcore_axis_name="s")
  cp = pltpu.CompilerParams(use_tc_tiling_on_sc=True, needs_layout_passes=False)
  wide = jax.ShapeDtypeStruct((BATCH, 128), jnp.float32)

  sc = functools.partial(
      pl.kernel, mesh=mesh, compiler_params=cp,
      out_type=(wide, wide, wide),
      scratch_types=[
          pltpu.VMEM((_BPW,), jnp.int32),
          pltpu.VMEM((_BPW, 128), jnp.float32),
          pltpu.VMEM((EMB, _PIECE), jnp.float32),
          pltpu.VMEM((EMB, _PIECE), jnp.float32),
          pltpu.VMEM((_BPW // 128, 128), jnp.int32),
          pltpu.VMEM((EMB, 256), jnp.float32),
          pltpu.SemaphoreType.DMA,
          pltpu.SemaphoreType.DMA,
          pltpu.SemaphoreType.DMA,
      ],
  )(_sc_body)
  item_w, user_w, dur_w = sc(item_t, user_t, dur_t, sids_i, sids_u,
                             pos_i, pos_u, duration)

  bm = 2048
  grid = (BATCH // bm,)
  full = lambda shape: pl.BlockSpec(shape, lambda i: (0,) * len(shape))
  row = lambda w: pl.BlockSpec((bm, w), lambda i: (i, 0))
  out = pl.pallas_call(
      _mlp_body,
      grid=grid,
      in_specs=[
          row(128), row(128), row(128),
          full((3 * EMB, 128)),
          full((1, 128)),
          full((128, 64)),
          full((1, 64)),
          full((64, 32)),
          full((1, 32)),
          full((32, 2)),
          full((1, 2)),
      ],
      out_specs=pl.BlockSpec((bm, 2), lambda i: (i, 0)),
      out_shape=jax.ShapeDtypeStruct((BATCH, 2), jnp.float32),
  )(item_w, user_w, dur_w,
    W1, b1.reshape(1, 128), W2, b2.reshape(1, 64), W3, b3.reshape(1, 32),
    Wo, bo.reshape(1, 2))
  return out


# speedup vs baseline: 1.0883x; 1.0883x over previous
import functools

import jax
import jax.numpy as jnp
from jax import lax
from jax.experimental import pallas as pl
from jax.experimental.pallas import tpu as pltpu
from jax.experimental.pallas import tpu_sc as plsc

BATCH = 16384
EMB = 32
_NC = 2
_NS = 16
_NW = _NC * _NS
_BPW = BATCH // _NW
_NG = _BPW // 16
_PIECE = 512
_TCOLS = 1000001
_TPAD = 1000064


def _stream_table(tab, ids_v, obuf, win0, win1, sem0, sem1):
  gmin = []
  gmax = []
  for g in range(_NG):
    idv = ids_v[pl.ds(g * 16, 16)]
    gmin.append(jnp.min(idv))
    gmax.append(jnp.max(idv))
  lo_all = jnp.minimum(functools.reduce(jnp.minimum, gmin), _TPAD - _PIECE)
  base = (lo_all // 128) * 128
  hi_all = functools.reduce(jnp.maximum, gmax)
  npieces = (hi_all - base) // _PIECE + 1

  def piece_start(p):
    return pl.multiple_of(
        jnp.minimum(base + p * _PIECE, _TPAD - _PIECE), 128)

  def fire(p, win, sem):
    pltpu.async_copy(tab.at[:, pl.ds(piece_start(p), _PIECE)], win, sem)

  def wait(p, win, sem):
    pltpu.make_async_copy(
        tab.at[:, pl.ds(piece_start(p), _PIECE)], win, sem).wait()

  def process(p, win):
    lo = piece_start(p)
    for g in range(_NG):
      @pl.when((gmin[g] < lo + _PIECE) & (gmax[g] >= lo))
      def _(g=g, win=win, lo=lo):
        idv = ids_v[pl.ds(g * 16, 16)]
        col = idv - lo
        msk = (col >= 0) & (col < _PIECE)
        cols = jnp.where(msk, col, 0)
        rows = lax.iota(jnp.int32, 16) + g * 16

        def kbody(k, c2):
          kv = jnp.full((16,), 0, jnp.int32) + k
          v = plsc.load_gather(win, [kv, cols], mask=msk)
          plsc.store_scatter(obuf, [rows, kv], v, mask=msk)
          return c2
        lax.fori_loop(0, EMB, kbody, 0)

  fire(0, win0, sem0)

  def body2(q, carry):
    del carry
    p0 = 2 * q
    p1 = p0 + 1

    @pl.when(p1 < npieces)
    def _():
      fire(p1, win1, sem1)

    wait(p0, win0, sem0)
    process(p0, win0)

    @pl.when(p1 < npieces)
    def _():
      @pl.when(p1 + 1 < npieces)
      def _():
        fire(p1 + 1, win0, sem0)
      wait(p1, win1, sem1)
      process(p1, win1)
    return 0

  lax.fori_loop(0, (npieces + 1) // 2, body2, 0)


def _sc_body(item_t, user_t, dur_t, sids_i, sids_u, pos_i, pos_u, dur_id,
             item_out, user_out, dur_out,
             ids_v, obuf, win0, win1, pos_v, dtab,
             sem0, sem1, ssem):
  wid = lax.axis_index("s") * _NC + lax.axis_index("c")
  sl = pl.ds(wid * _BPW, _BPW)

  def scatter_rows(out):
    copies = [
        pltpu.async_copy(obuf.at[pl.ds(c * 128, 128)],
                         out.at[pos_v.at[c]], ssem)
        for c in range(_BPW // 128)
    ]
    for cp in copies:
      cp.wait()

  pltpu.sync_copy(dur_t, dtab)
  pltpu.sync_copy(dur_id.at[sl], ids_v)
  for g in range(_NG):
    idv = ids_v[pl.ds(g * 16, 16)]
    rows = lax.iota(jnp.int32, 16) + g * 16

    def kbody(k, c2, idv=idv, rows=rows):
      kv = jnp.full((16,), 0, jnp.int32) + k
      v = plsc.load_gather(dtab, [kv, idv])
      plsc.store_scatter(obuf, [rows, kv], v)
      return c2
    lax.fori_loop(0, EMB, kbody, 0)
  pltpu.sync_copy(obuf, dur_out.at[sl])

  for tab, sids, pos3, out in ((item_t, sids_i, pos_i, item_out),
                               (user_t, sids_u, pos_u, user_out)):
    pltpu.sync_copy(sids.at[sl], ids_v)
    pltpu.sync_copy(pos3.at[wid], pos_v)
    _stream_table(tab, ids_v, obuf, win0, win1, sem0, sem1)
    scatter_rows(out)


def _mlp_body(item_ref, user_ref, dur_ref, w1_ref, b1_ref, w2_ref, b2_ref,
              w3_ref, b3_ref, wo_ref, bo_ref, out_ref):
  f32 = jnp.float32
  h = jnp.dot(item_ref[:, 0:EMB], w1_ref[0:EMB, :], preferred_element_type=f32)
  h += jnp.dot(user_ref[:, 0:EMB], w1_ref[EMB:2 * EMB, :], preferred_element_type=f32)
  h += jnp.dot(dur_ref[:, 0:EMB], w1_ref[2 * EMB:3 * EMB, :], preferred_element_type=f32)
  h = jnp.maximum(h + b1_ref[...], 0.0)
  h = jnp.maximum(jnp.dot(h, w2_ref[...], preferred_element_type=f32) + b2_ref[...], 0.0)
  h = jnp.maximum(jnp.dot(h, w3_ref[...], preferred_element_type=f32) + b3_ref[...], 0.0)
  z = jnp.dot(h, wo_ref[...], preferred_element_type=f32) + bo_ref[...]
  out_ref[...] = 1.0 / (1.0 + jnp.exp(-z))


def kernel(user_id, item_id, duration, is_training, item_table, user_table,
           dur_table, W1, b1, W2, b2, W3, b3, Wo, bo):
  del is_training

  item_id = item_id.astype(jnp.int32)
  user_id = user_id.astype(jnp.int32)
  duration = duration.astype(jnp.int32)

  item_t = item_table.T
  user_t = user_table.T
  dur_t = jnp.pad(dur_table.T, ((0, 0), (0, 56)))

  iota = lax.iota(jnp.int32, BATCH)

  def prep(ids):
    sids, pos = lax.sort([ids, iota], num_keys=1)
    return sids, pos.reshape(_NW, _BPW // 128, 128)

  sids_i, pos_i = prep(item_id)
  sids_u, pos_u = prep(user_id)

  mesh = plsc.VectorSubcoreMesh(core_axis_name="c", subcore_axis_name="s")
  cp = pltpu.CompilerParams(use_tc_tiling_on_sc=True, needs_layout_passes=False)
  wide = jax.ShapeDtypeStruct((BATCH, 128), jnp.float32)

  sc = functools.partial(
      pl.kernel, mesh=mesh, compiler_params=cp,
      out_type=(wide, wide, wide),
      scratch_types=[
          pltpu.VMEM((_BPW,), jnp.int32),
          pltpu.VMEM((_BPW, 128), jnp.float32),
          pltpu.VMEM((EMB, _PIECE), jnp.float32),
          pltpu.VMEM((EMB, _PIECE), jnp.float32),
          pltpu.VMEM((_BPW // 128, 128), jnp.int32),
          pltpu.VMEM((EMB, 256), jnp.float32),
          pltpu.SemaphoreType.DMA,
          pltpu.SemaphoreType.DMA,
          pltpu.SemaphoreType.DMA,
      ],
  )(_sc_body)
  item_w, user_w, dur_w = sc(item_t, user_t, dur_t, sids_i, sids_u,
                             pos_i, pos_u, duration)

  bm = 2048
  grid = (BATCH // bm,)
  full = lambda shape: pl.BlockSpec(shape, lambda i: (0,) * len(shape))
  row = lambda w: pl.BlockSpec((bm, w), lambda i: (i, 0))
  out = pl.pallas_call(
      _mlp_body,
      grid=grid,
      in_specs=[
          row(128), row(128), row(128),
          full((3 * EMB, 128)),
          full((1, 128)),
          full((128, 64)),
          full((1, 64)),
          full((64, 32)),
          full((1, 32)),
          full((32, 2)),
          full((1, 2)),
      ],
      out_specs=pl.BlockSpec((bm, 2), lambda i: (i, 0)),
      out_shape=jax.ShapeDtypeStruct((BATCH, 2), jnp.float32),
  )(item_w, user_w, dur_w,
    W1, b1.reshape(1, 128), W2, b2.reshape(1, 64), W3, b3.reshape(1, 32),
    Wo, bo.reshape(1, 2))
  return out


# speedup vs baseline: 1.0966x; 1.0076x over previous
import functools

import jax
import jax.numpy as jnp
from jax import lax
from jax.experimental import pallas as pl
from jax.experimental.pallas import tpu as pltpu
from jax.experimental.pallas import tpu_sc as plsc

BATCH = 16384
EMB = 32
_NC = 2
_NS = 16
_NW = _NC * _NS
_BPW = BATCH // _NW
_NG = _BPW // 16
_PIECE = 512
_TCOLS = 1000001
_TPAD = 1000064


def _stream_table(tab, ids_v, obuf, win0, win1, sem0, sem1):
  gmin = []
  gmax = []
  for g in range(_NG):
    idv = ids_v[pl.ds(g * 16, 16)]
    gmin.append(jnp.min(idv))
    gmax.append(jnp.max(idv))
  lo_all = jnp.minimum(functools.reduce(jnp.minimum, gmin), _TPAD - _PIECE)
  base = (lo_all // 128) * 128
  hi_all = functools.reduce(jnp.maximum, gmax)
  npieces = (hi_all - base) // _PIECE + 1

  def piece_start(p):
    return pl.multiple_of(
        jnp.minimum(base + p * _PIECE, _TPAD - _PIECE), 128)

  def fire(p, win, sem):
    pltpu.async_copy(tab.at[:, pl.ds(piece_start(p), _PIECE)], win, sem)

  def wait(p, win, sem):
    pltpu.make_async_copy(
        tab.at[:, pl.ds(piece_start(p), _PIECE)], win, sem).wait()

  def process(p, win):
    lo = piece_start(p)
    for g in range(_NG):
      @pl.when((gmin[g] < lo + _PIECE) & (gmax[g] >= lo))
      def _(g=g, win=win, lo=lo):
        idv = ids_v[pl.ds(g * 16, 16)]
        col = idv - lo
        msk = (col >= 0) & (col < _PIECE)
        cols = jnp.where(msk, col, 0)
        rows = lax.iota(jnp.int32, 16) + g * 16

        def kbody(k, c2):
          kv = jnp.full((16,), 0, jnp.int32) + k
          v = plsc.load_gather(win, [kv, cols], mask=msk)
          plsc.store_scatter(obuf, [rows, kv], v, mask=msk)
          return c2
        lax.fori_loop(0, EMB, kbody, 0)

  fire(0, win0, sem0)

  def body2(q, carry):
    del carry
    p0 = 2 * q
    p1 = p0 + 1

    @pl.when(p1 < npieces)
    def _():
      fire(p1, win1, sem1)

    wait(p0, win0, sem0)
    process(p0, win0)

    @pl.when(p1 < npieces)
    def _():
      @pl.when(p1 + 1 < npieces)
      def _():
        fire(p1 + 1, win0, sem0)
      wait(p1, win1, sem1)
      process(p1, win1)
    return 0

  lax.fori_loop(0, (npieces + 1) // 2, body2, 0)


def _sc_body(item_t, user_t, dur_t, sids_i, sids_u, pos_i, pos_u, dur_id,
             item_out, user_out, dur_out,
             ids_iv, ids_uv, ids_dv, obuf, win0, win1, pos_iv, pos_uv, dtab,
             sem0, sem1, ssem):
  wid = lax.axis_index("s") * _NC + lax.axis_index("c")
  sl = pl.ds(wid * _BPW, _BPW)

  def scatter_rows(out, pos_v):
    copies = [
        pltpu.async_copy(obuf.at[pl.ds(c * 128, 128)],
                         out.at[pos_v.at[c]], ssem)
        for c in range(_BPW // 128)
    ]
    for cp in copies:
      cp.wait()

  for tab, sids, pos3, out, ids_v, pos_v in (
      (item_t, sids_i, pos_i, item_out, ids_iv, pos_iv),
      (user_t, sids_u, pos_u, user_out, ids_uv, pos_uv)):
    pltpu.sync_copy(sids.at[sl], ids_v)
    pltpu.sync_copy(pos3.at[wid], pos_v)
    _stream_table(tab, ids_v, obuf, win0, win1, sem0, sem1)
    scatter_rows(out, pos_v)

  pltpu.sync_copy(dur_t, dtab)
  pltpu.sync_copy(dur_id.at[sl], ids_dv)
  for g in range(_NG):
    idv = ids_dv[pl.ds(g * 16, 16)]
    rows = lax.iota(jnp.int32, 16) + g * 16

    def kbody(k, c2, idv=idv, rows=rows):
      kv = jnp.full((16,), 0, jnp.int32) + k
      v = plsc.load_gather(dtab, [kv, idv])
      plsc.store_scatter(obuf, [rows, kv], v)
      return c2
    lax.fori_loop(0, EMB, kbody, 0)
  pltpu.sync_copy(obuf, dur_out.at[sl])


def _mlp_body(item_ref, user_ref, dur_ref, w1_ref, b1_ref, w2_ref, b2_ref,
              w3_ref, b3_ref, wo_ref, bo_ref, out_ref):
  f32 = jnp.float32
  h = jnp.dot(item_ref[:, 0:EMB], w1_ref[0:EMB, :], preferred_element_type=f32)
  h += jnp.dot(user_ref[:, 0:EMB], w1_ref[EMB:2 * EMB, :], preferred_element_type=f32)
  h += jnp.dot(dur_ref[:, 0:EMB], w1_ref[2 * EMB:3 * EMB, :], preferred_element_type=f32)
  h = jnp.maximum(h + b1_ref[...], 0.0)
  h = jnp.maximum(jnp.dot(h, w2_ref[...], preferred_element_type=f32) + b2_ref[...], 0.0)
  h = jnp.maximum(jnp.dot(h, w3_ref[...], preferred_element_type=f32) + b3_ref[...], 0.0)
  z = jnp.dot(h, wo_ref[...], preferred_element_type=f32) + bo_ref[...]
  out_ref[...] = 1.0 / (1.0 + jnp.exp(-z))


def kernel(user_id, item_id, duration, is_training, item_table, user_table,
           dur_table, W1, b1, W2, b2, W3, b3, Wo, bo):
  del is_training

  item_id = item_id.astype(jnp.int32)
  user_id = user_id.astype(jnp.int32)
  duration = duration.astype(jnp.int32)

  item_t = item_table.T
  user_t = user_table.T
  dur_t = jnp.pad(dur_table.T, ((0, 0), (0, 56)))

  iota = lax.iota(jnp.int32, BATCH)

  def prep(ids):
    sids, pos = lax.sort([ids, iota], num_keys=1)
    return sids, pos.reshape(_NW, _BPW // 128, 128)

  sids_i, pos_i = prep(item_id)
  sids_u, pos_u = prep(user_id)

  mesh = plsc.VectorSubcoreMesh(core_axis_name="c", subcore_axis_name="s")
  cp = pltpu.CompilerParams(use_tc_tiling_on_sc=True, needs_layout_passes=False)
  wide = jax.ShapeDtypeStruct((BATCH, 128), jnp.float32)

  sc = functools.partial(
      pl.kernel, mesh=mesh, compiler_params=cp,
      out_type=(wide, wide, wide),
      scratch_types=[
          pltpu.VMEM((_BPW,), jnp.int32),
          pltpu.VMEM((_BPW,), jnp.int32),
          pltpu.VMEM((_BPW,), jnp.int32),
          pltpu.VMEM((_BPW, 128), jnp.float32),
          pltpu.VMEM((EMB, _PIECE), jnp.float32),
          pltpu.VMEM((EMB, _PIECE), jnp.float32),
          pltpu.VMEM((_BPW // 128, 128), jnp.int32),
          pltpu.VMEM((_BPW // 128, 128), jnp.int32),
          pltpu.VMEM((EMB, 256), jnp.float32),
          pltpu.SemaphoreType.DMA,
          pltpu.SemaphoreType.DMA,
          pltpu.SemaphoreType.DMA,
      ],
  )(_sc_body)
  item_w, user_w, dur_w = sc(item_t, user_t, dur_t, sids_i, sids_u,
                             pos_i, pos_u, duration)

  bm = 2048
  grid = (BATCH // bm,)
  full = lambda shape: pl.BlockSpec(shape, lambda i: (0,) * len(shape))
  row = lambda w: pl.BlockSpec((bm, w), lambda i: (i, 0))
  out = pl.pallas_call(
      _mlp_body,
      grid=grid,
      in_specs=[
          row(128), row(128), row(128),
          full((3 * EMB, 128)),
          full((1, 128)),
          full((128, 64)),
          full((1, 64)),
          full((64, 32)),
          full((1, 32)),
          full((32, 2)),
          full((1, 2)),
      ],
      out_specs=pl.BlockSpec((bm, 2), lambda i: (i, 0)),
      out_shape=jax.ShapeDtypeStruct((BATCH, 2), jnp.float32),
  )(item_w, user_w, dur_w,
    W1, b1.reshape(1, 128), W2, b2.reshape(1, 64), W3, b3.reshape(1, 32),
    Wo, bo.reshape(1, 2))
  return out
